# baseline (speedup 1.0000x reference)
import jax
import jax.numpy as jnp
from jax.experimental import pallas as pl
from jax.experimental.pallas import tpu as pltpu

_EPS = 0.001
_C = 256
_GROUP = 128


def _factor(gram, srow, n_total, wt_s, bias_s, a_s, uw_s):
    c = _C
    gsz = _GROUP
    nf = jnp.float32(n_total)
    m = srow / nf
    outer = jax.lax.dot_general(
        m, m, (((0,), (0,)), ((), ())),
        preferred_element_type=jnp.float32)
    cov = (gram - nf * outer) / (nf - 1.0)
    lane_c = jax.lax.broadcasted_iota(jnp.int32, (1, c), 1)
    row_c = jax.lax.broadcasted_iota(jnp.int32, (c, 1), 0)
    eye = (lane_c == row_c).astype(jnp.float32)
    a_s[...] = (1.0 - _EPS) * cov + _EPS * eye
    uw_s[...] = jnp.zeros_like(uw_s)

    subg = jax.lax.broadcasted_iota(jnp.int32, (gsz, 1), 0)
    lane_g = jax.lax.broadcasted_iota(jnp.int32, (1, gsz), 1)
    eye_g = (lane_g == subg).astype(jnp.float32)

    for grp in range(c // gsz):
        j0 = grp * gsz
        arows = a_s[j0:j0 + gsz, :]
        eg = (lane_c == subg + j0).astype(jnp.float32)
        if grp == 0:
            slab_s, slab_p = arows, eg
        else:
            ucols = uw_s[:, j0:j0 + gsz]
            corr = jax.lax.dot_general(
                ucols, uw_s[...], (((0,), (0,)), ((), ())),
                preferred_element_type=jnp.float32)
            slab_s = arows - corr[:, 0:c]
            slab_p = eg - corr[:, c:2 * c]
        dblk = jax.lax.slice(slab_s, (0, j0), (gsz, j0 + gsz))
        mmat = jnp.concatenate([dblk, eye_g], axis=1)
        rs = []
        for t in range(gsz):
            colt = jax.lax.slice(mmat, (0, t), (gsz, t + 1))
            dt = jax.lax.slice(colt, (t, 0), (t + 1, 1))
            rsq = jax.lax.rsqrt(dt)
            rc = rsq * rsq
            prow = jax.lax.slice(mmat, (t, 0), (t + 1, 2 * gsz))
            coef = jnp.where(subg > t, colt, 0.0) * rc
            mmat = mmat - coef * prow
            rs.append(rsq)
        rvec = jnp.concatenate(rs, axis=0)
        linv = rvec * mmat[:, gsz:2 * gsz]
        ugwg = jax.lax.dot_general(
            linv, jnp.concatenate([slab_s, slab_p], axis=1),
            (((1,), (0,)), ((), ())),
            preferred_element_type=jnp.float32)
        ug = jnp.where(lane_c >= subg + j0, ugwg[:, 0:c], 0.0)
        wg = jnp.where(lane_c <= subg + j0, ugwg[:, c:2 * c], 0.0)
        uw_s[j0:j0 + gsz, :] = jnp.concatenate([ug, wg], axis=1)

    wv = uw_s[:, c:2 * c]
    wt_s[...] = wv.T
    bias_s[...] = jax.lax.dot_general(
        m, wv, (((1,), (1,)), ((), ())),
        preferred_element_type=jnp.float32)


def _fused_kernel(nb, n_total, x_ref, o_ref,
                  gram_s, sum_s, a_s, uw_s, wt_s, bias_s):
    j = pl.program_id(0)

    @pl.when(j == 0)
    def _():
        gram_s[...] = jnp.zeros_like(gram_s)
        sum_s[...] = jnp.zeros_like(sum_s)

    @pl.when(j < nb)
    def _():
        blk = x_ref[...]
        gram_s[...] += jax.lax.dot_general(
            blk, blk, (((0,), (0,)), ((), ())),
            preferred_element_type=jnp.float32)
        sum_s[...] += jnp.sum(blk, axis=0, keepdims=True)

    @pl.when(j == nb)
    def _():
        _factor(gram_s[...], sum_s[...], n_total, wt_s, bias_s, a_s, uw_s)

    @pl.when(j > nb)
    def _():
        o_ref[...] = jax.lax.dot_general(
            x_ref[...], wt_s[...], (((1,), (0,)), ((), ())),
            preferred_element_type=jnp.float32) - bias_s[...]


def kernel(x):
    b, w, h, c = x.shape
    n = b * w * h
    x2 = x.reshape(n, c)
    bn = 4096
    nb = n // bn

    def x_map(j):
        return (jnp.where(j < nb, j, jnp.maximum(j - (nb + 1), 0)), 0)

    def o_map(j):
        return (jnp.maximum(j - (nb + 1), 0), 0)

    out2 = pl.pallas_call(
        lambda x_ref, o_ref, *scr: _fused_kernel(nb, n, x_ref, o_ref, *scr),
        grid=(2 * nb + 1,),
        in_specs=[pl.BlockSpec((bn, c), x_map)],
        out_specs=pl.BlockSpec((bn, c), o_map),
        out_shape=jax.ShapeDtypeStruct((n, c), jnp.float32),
        scratch_shapes=[
            pltpu.VMEM((c, c), jnp.float32),
            pltpu.VMEM((1, c), jnp.float32),
            pltpu.VMEM((c, c), jnp.float32),
            pltpu.VMEM((c, 2 * c), jnp.float32),
            pltpu.VMEM((c, c), jnp.float32),
            pltpu.VMEM((1, c), jnp.float32),
        ],
        compiler_params=pltpu.CompilerParams(
            dimension_semantics=("arbitrary",),
            vmem_limit_bytes=50 * 1024 * 1024),
        name="decor_fused",
    )(x2)

    return out2.reshape(b, w, h, c)


# speedup vs baseline: 1.0680x; 1.0680x over previous
import jax
import jax.numpy as jnp
from jax.experimental import pallas as pl
from jax.experimental.pallas import tpu as pltpu

_EPS = 0.001
_C = 256
_GROUP = 128


def _factor(gram, srow, n_total, wt_s, bias_s, a_s, uw_s):
    c = _C
    gsz = _GROUP
    nf = jnp.float32(n_total)
    m = srow / nf
    outer = jax.lax.dot_general(
        m, m, (((0,), (0,)), ((), ())),
        preferred_element_type=jnp.float32)
    cov = (gram - nf * outer) / (nf - 1.0)
    lane_c = jax.lax.broadcasted_iota(jnp.int32, (1, c), 1)
    row_c = jax.lax.broadcasted_iota(jnp.int32, (c, 1), 0)
    eye = (lane_c == row_c).astype(jnp.float32)
    a_s[...] = (1.0 - _EPS) * cov + _EPS * eye
    uw_s[...] = jnp.zeros_like(uw_s)

    subg = jax.lax.broadcasted_iota(jnp.int32, (gsz, 1), 0)
    lane_g = jax.lax.broadcasted_iota(jnp.int32, (1, gsz), 1)
    eye_g = (lane_g == subg).astype(jnp.float32)

    for grp in range(c // gsz):
        j0 = grp * gsz
        arows = a_s[j0:j0 + gsz, :]
        eg = (lane_c == subg + j0).astype(jnp.float32)
        if grp == 0:
            slab_s, slab_p = arows, eg
        else:
            ucols = uw_s[:, j0:j0 + gsz]
            corr = jax.lax.dot_general(
                ucols, uw_s[...], (((0,), (0,)), ((), ())),
                preferred_element_type=jnp.float32)
            slab_s = arows - corr[:, 0:c]
            slab_p = eg - corr[:, c:2 * c]
        dblk = jax.lax.slice(slab_s, (0, j0), (gsz, j0 + gsz))
        mmat = jnp.concatenate([dblk, eye_g], axis=1)
        rs = []
        for t in range(gsz):
            colt = jax.lax.slice(mmat, (0, t), (gsz, t + 1))
            dt = jax.lax.slice(colt, (t, 0), (t + 1, 1))
            rsq = jax.lax.rsqrt(dt)
            rc = rsq * rsq
            prow = jax.lax.slice(mmat, (t, 0), (t + 1, 2 * gsz))
            coef = jnp.where(subg > t, colt, 0.0) * rc
            mmat = mmat - coef * prow
            rs.append(rsq)
        rvec = jnp.concatenate(rs, axis=0)
        linv = rvec * mmat[:, gsz:2 * gsz]
        ugwg = jax.lax.dot_general(
            linv, jnp.concatenate([slab_s, slab_p], axis=1),
            (((1,), (0,)), ((), ())),
            preferred_element_type=jnp.float32)
        ug = jnp.where(lane_c >= subg + j0, ugwg[:, 0:c], 0.0)
        wg = jnp.where(lane_c <= subg + j0, ugwg[:, c:2 * c], 0.0)
        uw_s[j0:j0 + gsz, :] = jnp.concatenate([ug, wg], axis=1)

    wv = uw_s[:, c:2 * c]
    wt_s[...] = wv.T
    bias_s[...] = jax.lax.dot_general(
        m, wv, (((1,), (1,)), ((), ())),
        preferred_element_type=jnp.float32)


def _fused_kernel(nb, n_total, x_ref, o_ref,
                  gram_s, sum_s, a_s, uw_s, wt_s, bias_s):
    j = pl.program_id(0)

    @pl.when(j == 0)
    def _():
        gram_s[...] = jnp.zeros_like(gram_s)
        sum_s[...] = jnp.zeros_like(sum_s)

    @pl.when(j < nb)
    def _():
        blk = x_ref[...]
        gram_s[...] += jax.lax.dot_general(
            blk, blk, (((0,), (0,)), ((), ())),
            preferred_element_type=jnp.float32)
        sum_s[...] += jnp.sum(blk, axis=0, keepdims=True)

    @pl.when(j == nb)
    def _():
        _factor(gram_s[...], sum_s[...], n_total, wt_s, bias_s, a_s, uw_s)

    @pl.when(j > nb)
    def _():
        o_ref[...] = jax.lax.dot_general(
            x_ref[...], wt_s[...], (((1,), (0,)), ((), ())),
            preferred_element_type=jnp.float32) - bias_s[...]


def kernel(x):
    b, w, h, c = x.shape
    n = b * w * h
    x2 = x.reshape(n, c)
    bn = 8192
    nb = n // bn

    def x_map(j):
        return (jnp.where(j < nb, j, jnp.maximum(j - (nb + 1), 0)), 0)

    def o_map(j):
        return (jnp.maximum(j - (nb + 1), 0), 0)

    out2 = pl.pallas_call(
        lambda x_ref, o_ref, *scr: _fused_kernel(nb, n, x_ref, o_ref, *scr),
        grid=(2 * nb + 1,),
        in_specs=[pl.BlockSpec((bn, c), x_map)],
        out_specs=pl.BlockSpec((bn, c), o_map),
        out_shape=jax.ShapeDtypeStruct((n, c), jnp.float32),
        scratch_shapes=[
            pltpu.VMEM((c, c), jnp.float32),
            pltpu.VMEM((1, c), jnp.float32),
            pltpu.VMEM((c, c), jnp.float32),
            pltpu.VMEM((c, 2 * c), jnp.float32),
            pltpu.VMEM((c, c), jnp.float32),
            pltpu.VMEM((1, c), jnp.float32),
        ],
        compiler_params=pltpu.CompilerParams(
            dimension_semantics=("arbitrary",),
            vmem_limit_bytes=50 * 1024 * 1024),
        name="decor_fused",
    )(x2)

    return out2.reshape(b, w, h, c)
